# baseline (speedup 1.0000x reference)
import dataclasses
import functools

import jax
import jax.numpy as jnp
from jax import lax
from jax.experimental import pallas as pl
from jax.experimental.pallas import tpu as pltpu
from jax.experimental.pallas import tpu_sc as plsc

N_NODES = 10000
N_EDGES = 320000
D = 128

NC = 2
NS = 16
NW = NC * NS
E_PER_TILE = N_EDGES // NW
CHUNK = 128
N_CHUNKS = 79
E_TILE = N_CHUNKS * CHUNK
E_PAD = NW * E_TILE
N_PAD = 10240
ROWS_PER_TILE = N_PAD // NS
BR = 1024

_mesh = plsc.VectorSubcoreMesh(
    core_axis_name="c", subcore_axis_name="s", num_cores=NC, num_subcores=NS
)

_cp = pltpu.CompilerParams()
if "needs_layout_passes" in pltpu.CompilerParams.__dataclass_fields__:
    _cp = dataclasses.replace(_cp, needs_layout_passes=False)


def _deg_partials(dst_flat):
    @functools.partial(
        pl.kernel,
        out_type=jax.ShapeDtypeStruct((NC, NS, N_PAD), jnp.float32),
        mesh=_mesh,
        scratch_types=[
            pltpu.VMEM((E_PER_TILE,), jnp.int32),
            pltpu.VMEM((N_PAD,), jnp.float32),
        ],
        compiler_params=_cp,
    )
    def k(dst_hbm, degp_hbm, didx, hist):
        cid = lax.axis_index("c")
        sid = lax.axis_index("s")
        ebase = (cid * NS + sid) * E_PER_TILE
        pltpu.sync_copy(dst_hbm.at[pl.ds(ebase, E_PER_TILE)], didx)

        @pl.loop(0, N_PAD, step=16)
        def _(i):
            hist[pl.ds(i, 16)] = jnp.zeros((16,), jnp.float32)

        ones16 = jnp.ones((16,), jnp.float32)

        @pl.loop(0, E_PER_TILE, step=16)
        def _(k):
            plsc.addupdate_scatter(hist, [didx[pl.ds(k, 16)]], ones16)

        pltpu.sync_copy(hist, degp_hbm.at[cid, sid])

    return k(dst_flat)


def _scaled_linear(x_pad, W, degp):
    def body(x_ref, w_ref, d_ref, o_ref, dv_ref):
        h = lax.dot_general(
            x_ref[...],
            w_ref[...],
            (((1,), (1,)), ((), ())),
            preferred_element_type=jnp.float32,
            precision=lax.Precision.HIGHEST,
        )
        deg = jnp.sum(d_ref[0] + d_ref[1], axis=0)
        dinv = jnp.where(deg > 0, lax.rsqrt(jnp.maximum(deg, 1e-12)), 0.0)
        o_ref[...] = h * dinv[:, None]
        dv_ref[...] = dinv[:, None]

    return pl.pallas_call(
        body,
        grid=(N_PAD // BR,),
        in_specs=[
            pl.BlockSpec((BR, D), lambda i: (i, 0)),
            pl.BlockSpec((D, D), lambda i: (0, 0)),
            pl.BlockSpec((NC, NS, BR), lambda i: (0, 0, i)),
        ],
        out_specs=[
            pl.BlockSpec((BR, D), lambda i: (i, 0)),
            pl.BlockSpec((BR, 1), lambda i: (i, 0)),
        ],
        out_shape=[
            jax.ShapeDtypeStruct((N_PAD, D), jnp.float32),
            jax.ShapeDtypeStruct((N_PAD, 1), jnp.float32),
        ],
    )(x_pad, W, degp)


def _gather_scatter(hs, src_pad, dst_pad):
    @functools.partial(
        pl.kernel,
        out_type=jax.ShapeDtypeStruct((NC, N_PAD, D), jnp.float32),
        mesh=_mesh,
        scratch_types=[
            pltpu.VMEM((CHUNK,), jnp.int32),
            pltpu.VMEM((CHUNK,), jnp.int32),
            pltpu.VMEM((CHUNK,), jnp.int32),
            pltpu.VMEM((CHUNK,), jnp.int32),
            pltpu.VMEM((CHUNK, D), jnp.float32),
            pltpu.VMEM((CHUNK, D), jnp.float32),
            pltpu.VMEM_SHARED((N_PAD, D), jnp.float32),
            pltpu.SemaphoreType.DMA,
            pltpu.SemaphoreType.DMA,
            pltpu.SemaphoreType.DMA,
            pltpu.SemaphoreType.DMA,
            pltpu.SemaphoreType.DMA,
            pltpu.SemaphoreType.DMA,
        ],
    )
    def k(hs_hbm, src_hbm, dst_hbm, acc_hbm,
          sidxa, sidxb, didxa, didxb, rowsa, rowsb, acc_sh,
          gsema, gsemb, isema, isemb, dsema, dsemb):
        cid = lax.axis_index("c")
        sid = lax.axis_index("s")
        wid = cid * NS + sid
        row0 = sid * ROWS_PER_TILE
        ebase = wid * E_TILE

        def iload(hbm, c, buf, sem):
            c = jnp.minimum(c, N_CHUNKS - 1)
            pltpu.async_copy(hbm.at[pl.ds(ebase + c * CHUNK, CHUNK)], buf, sem)

        def iwait(hbm, buf, sem):
            pltpu.make_async_copy(hbm.at[pl.ds(ebase, CHUNK)], buf, sem).wait()

        def gstart(buf, idx, sem):
            pltpu.async_copy(hs_hbm.at[idx], buf, sem)

        def gwait(buf, idx, sem):
            pltpu.make_async_copy(hs_hbm.at[idx], buf, sem).wait()

        def scatter(buf, idx):
            pltpu.sync_copy(buf, acc_sh.at[idx], add=True)

        @pl.loop(0, CHUNK)
        def _(i):
            @pl.loop(0, D, step=16)
            def _(j):
                rowsa[i, pl.ds(j, 16)] = jnp.zeros((16,), jnp.float32)

        @pl.loop(0, ROWS_PER_TILE, step=CHUNK)
        def _(r):
            @pl.loop(0, CHUNK, step=16)
            def _(i):
                sidxa[pl.ds(i, 16)] = lax.iota(jnp.int32, 16) + (row0 + r + i)
            pltpu.sync_copy(rowsa, acc_sh.at[sidxa])
        plsc.subcore_barrier()

        pltpu.sync_copy(src_hbm.at[pl.ds(ebase, CHUNK)], sidxa)
        gstart(rowsa, sidxa, gsema)
        iload(dst_hbm, 0, didxa, dsema)
        iload(src_hbm, 1, sidxb, isemb)
        iload(dst_hbm, 1, didxb, dsemb)

        @pl.loop(0, N_CHUNKS - 1, step=2)
        def _(j):
            gwait(rowsa, sidxa, gsema)
            iload(src_hbm, j + 2, sidxa, isema)
            iwait(src_hbm, sidxb, isemb)
            gstart(rowsb, sidxb, gsemb)
            iwait(dst_hbm, didxa, dsema)
            scatter(rowsa, didxa)
            iload(dst_hbm, j + 2, didxa, dsema)
            gwait(rowsb, sidxb, gsemb)
            iload(src_hbm, j + 3, sidxb, isemb)
            iwait(src_hbm, sidxa, isema)
            gstart(rowsa, sidxa, gsema)
            iwait(dst_hbm, didxb, dsemb)
            scatter(rowsb, didxb)
            iload(dst_hbm, j + 3, didxb, dsemb)

        gwait(rowsa, sidxa, gsema)
        iwait(dst_hbm, didxa, dsema)
        scatter(rowsa, didxa)
        iwait(src_hbm, sidxb, isemb)
        iwait(dst_hbm, didxb, dsemb)

        plsc.subcore_barrier()
        pltpu.sync_copy(
            acc_sh.at[pl.ds(row0, ROWS_PER_TILE)],
            acc_hbm.at[cid, pl.ds(row0, ROWS_PER_TILE)],
        )

    return k(hs, src_pad, dst_pad)


def _epilogue(accp, dinv):
    def body(a_ref, dv_ref, o_ref):
        o = (a_ref[0] + a_ref[1]) * dv_ref[...]
        o_ref[...] = jnp.maximum(o, 0.0) + o

    return pl.pallas_call(
        body,
        grid=(N_PAD // BR,),
        in_specs=[
            pl.BlockSpec((NC, BR, D), lambda i: (0, i, 0)),
            pl.BlockSpec((BR, 1), lambda i: (i, 0)),
        ],
        out_specs=pl.BlockSpec((BR, D), lambda i: (i, 0)),
        out_shape=jax.ShapeDtypeStruct((N_PAD, D), jnp.float32),
    )(accp, dinv)


@jax.jit
def kernel(x, edge_index, edge_attr, W):
    src_flat = edge_index[0]
    dst_flat = edge_index[1]
    x_pad = jnp.concatenate(
        [x, jnp.zeros((N_PAD - N_NODES, D), jnp.float32)], axis=0
    )
    fill = jnp.full((E_PAD - N_EDGES,), N_NODES, jnp.int32)
    src_pad = jnp.concatenate([src_flat, fill])
    dst_pad = jnp.concatenate([dst_flat, fill])
    degp = _deg_partials(dst_flat)
    hs, dinv = _scaled_linear(x_pad, W, degp)
    accp = _gather_scatter(hs, src_pad, dst_pad)
    return _epilogue(accp, dinv)[:N_NODES]


# speedup vs baseline: 1.7199x; 1.7199x over previous
import dataclasses
import functools

import jax
import jax.numpy as jnp
from jax import lax
from jax.experimental import pallas as pl
from jax.experimental.pallas import tpu as pltpu
from jax.experimental.pallas import tpu_sc as plsc

N_NODES = 10000
N_EDGES = 320000
D = 128

NC = 2
NS = 16
NW = NC * NS
E_PER_TILE = N_EDGES // NW
CHUNK = 80
N_CHUNKS = E_PER_TILE // CHUNK
N_PAD = 10240
ROWS_PER_TILE = N_PAD // NS
BR = 1024

_mesh = plsc.VectorSubcoreMesh(
    core_axis_name="c", subcore_axis_name="s", num_cores=NC, num_subcores=NS
)

_cp = pltpu.CompilerParams()
if "needs_layout_passes" in pltpu.CompilerParams.__dataclass_fields__:
    _cp = dataclasses.replace(_cp, needs_layout_passes=False)


def _deg_partials(dst_flat):
    @functools.partial(
        pl.kernel,
        out_type=jax.ShapeDtypeStruct((NC, NS, N_PAD), jnp.float32),
        mesh=_mesh,
        scratch_types=[
            pltpu.VMEM((E_PER_TILE,), jnp.int32),
            pltpu.VMEM((N_PAD,), jnp.float32),
        ],
        compiler_params=_cp,
    )
    def k(dst_hbm, degp_hbm, didx, hist):
        cid = lax.axis_index("c")
        sid = lax.axis_index("s")
        ebase = (cid * NS + sid) * E_PER_TILE
        pltpu.sync_copy(dst_hbm.at[pl.ds(ebase, E_PER_TILE)], didx)

        @pl.loop(0, N_PAD, step=16)
        def _(i):
            hist[pl.ds(i, 16)] = jnp.zeros((16,), jnp.float32)

        ones16 = jnp.ones((16,), jnp.float32)

        @pl.loop(0, E_PER_TILE, step=16)
        def _(k):
            plsc.addupdate_scatter(hist, [didx[pl.ds(k, 16)]], ones16)

        pltpu.sync_copy(hist, degp_hbm.at[cid, sid])

    return k(dst_flat)


def _scaled_linear(x_pad, W, degp):
    def body(x_ref, w_ref, d_ref, o_ref, dv_ref):
        h = lax.dot_general(
            x_ref[...],
            w_ref[...],
            (((1,), (1,)), ((), ())),
            preferred_element_type=jnp.float32,
            precision=lax.Precision.HIGHEST,
        )
        deg = jnp.sum(d_ref[0] + d_ref[1], axis=0)
        dinv = jnp.where(deg > 0, lax.rsqrt(jnp.maximum(deg, 1e-12)), 0.0)
        o_ref[...] = h * dinv[:, None]
        dv_ref[...] = dinv[:, None]

    return pl.pallas_call(
        body,
        grid=(N_PAD // BR,),
        in_specs=[
            pl.BlockSpec((BR, D), lambda i: (i, 0)),
            pl.BlockSpec((D, D), lambda i: (0, 0)),
            pl.BlockSpec((NC, NS, BR), lambda i: (0, 0, i)),
        ],
        out_specs=[
            pl.BlockSpec((BR, D), lambda i: (i, 0)),
            pl.BlockSpec((BR, 1), lambda i: (i, 0)),
        ],
        out_shape=[
            jax.ShapeDtypeStruct((N_PAD, D), jnp.float32),
            jax.ShapeDtypeStruct((N_PAD, 1), jnp.float32),
        ],
    )(x_pad, W, degp)


def _gather_scatter(hs, src_flat, dst3):
    @functools.partial(
        pl.kernel,
        out_type=jax.ShapeDtypeStruct((NC, N_PAD, D), jnp.float32),
        mesh=_mesh,
        scratch_types=[
            pltpu.VMEM((N_CHUNKS, CHUNK), jnp.int32),
            pltpu.VMEM((CHUNK,), jnp.int32),
            pltpu.VMEM((CHUNK,), jnp.int32),
            pltpu.VMEM((CHUNK, D), jnp.float32),
            pltpu.VMEM((CHUNK, D), jnp.float32),
            pltpu.VMEM_SHARED((N_PAD, D), jnp.float32),
            pltpu.SemaphoreType.DMA,
            pltpu.SemaphoreType.DMA,
            pltpu.SemaphoreType.DMA,
            pltpu.SemaphoreType.DMA,
        ],
    )
    def k(hs_hbm, src_hbm, dst_hbm, acc_hbm,
          didx, sidxa, sidxb, rowsa, rowsb, acc_sh, gsema, gsemb, isema, isemb):
        cid = lax.axis_index("c")
        sid = lax.axis_index("s")
        wid = cid * NS + sid
        row0 = sid * ROWS_PER_TILE
        ebase = wid * E_PER_TILE

        def sload(c, buf, sem):
            c = jnp.minimum(c, N_CHUNKS - 1)
            pltpu.async_copy(src_hbm.at[pl.ds(ebase + c * CHUNK, CHUNK)],
                             buf, sem)

        def swait(buf, sem):
            pltpu.make_async_copy(src_hbm.at[pl.ds(ebase, CHUNK)],
                                  buf, sem).wait()

        def gstart(buf, idx, sem):
            pltpu.async_copy(hs_hbm.at[idx], buf, sem)

        def gwait(buf, idx, sem):
            pltpu.make_async_copy(hs_hbm.at[idx], buf, sem).wait()

        def scatter(buf, c):
            pltpu.sync_copy(buf, acc_sh.at[didx.at[c]], add=True)

        @pl.loop(0, CHUNK)
        def _(i):
            @pl.loop(0, D, step=16)
            def _(j):
                rowsa[i, pl.ds(j, 16)] = jnp.zeros((16,), jnp.float32)

        @pl.loop(0, ROWS_PER_TILE, step=CHUNK)
        def _(r):
            @pl.loop(0, CHUNK, step=16)
            def _(i):
                sidxa[pl.ds(i, 16)] = lax.iota(jnp.int32, 16) + (row0 + r + i)
            pltpu.sync_copy(rowsa, acc_sh.at[sidxa])
        plsc.subcore_barrier()

        pltpu.sync_copy(dst_hbm.at[wid], didx)

        pltpu.sync_copy(src_hbm.at[pl.ds(ebase, CHUNK)], sidxa)
        gstart(rowsa, sidxa, gsema)
        sload(1, sidxb, isemb)

        @pl.loop(0, N_CHUNKS - 1, step=2)
        def _(j):
            gwait(rowsa, sidxa, gsema)
            sload(j + 2, sidxa, isema)
            swait(sidxb, isemb)
            gstart(rowsb, sidxb, gsemb)
            scatter(rowsa, j)
            gwait(rowsb, sidxb, gsemb)
            sload(j + 3, sidxb, isemb)
            swait(sidxa, isema)
            gstart(rowsa, sidxa, gsema)
            scatter(rowsb, j + 1)

        gwait(rowsa, sidxa, gsema)
        scatter(rowsa, N_CHUNKS - 1)
        swait(sidxb, isemb)

        plsc.subcore_barrier()
        pltpu.sync_copy(
            acc_sh.at[pl.ds(row0, ROWS_PER_TILE)],
            acc_hbm.at[cid, pl.ds(row0, ROWS_PER_TILE)],
        )

    return k(hs, src_flat, dst3)


def _epilogue(accp, dinv):
    def body(a_ref, dv_ref, o_ref):
        o = (a_ref[0] + a_ref[1]) * dv_ref[...]
        o_ref[...] = jnp.maximum(o, 0.0) + o

    return pl.pallas_call(
        body,
        grid=(N_PAD // BR,),
        in_specs=[
            pl.BlockSpec((NC, BR, D), lambda i: (0, i, 0)),
            pl.BlockSpec((BR, 1), lambda i: (i, 0)),
        ],
        out_specs=pl.BlockSpec((BR, D), lambda i: (i, 0)),
        out_shape=jax.ShapeDtypeStruct((N_PAD, D), jnp.float32),
    )(accp, dinv)


@jax.jit
def kernel(x, edge_index, edge_attr, W):
    src_flat = edge_index[0]
    dst_flat = edge_index[1]
    x_pad = jnp.concatenate(
        [x, jnp.zeros((N_PAD - N_NODES, D), jnp.float32)], axis=0
    )
    dst3 = dst_flat.reshape(NW, N_CHUNKS, CHUNK)
    degp = _deg_partials(dst_flat)
    hs, dinv = _scaled_linear(x_pad, W, degp)
    accp = _gather_scatter(hs, src_flat, dst3)
    return _epilogue(accp, dinv)[:N_NODES]


# speedup vs baseline: 1.7597x; 1.0231x over previous
import dataclasses
import functools

import jax
import jax.numpy as jnp
from jax import lax
from jax.experimental import pallas as pl
from jax.experimental.pallas import tpu as pltpu
from jax.experimental.pallas import tpu_sc as plsc

N_NODES = 10000
N_EDGES = 320000
D = 128

NC = 2
NS = 16
NW = NC * NS
E_PER_TILE = N_EDGES // NW
CHUNK = 80
N_CHUNKS = E_PER_TILE // CHUNK
N_PAD = 10240
ROWS_PER_TILE = N_PAD // NS
BR = 1024

_mesh = plsc.VectorSubcoreMesh(
    core_axis_name="c", subcore_axis_name="s", num_cores=NC, num_subcores=NS
)

_cp = pltpu.CompilerParams()
if "needs_layout_passes" in pltpu.CompilerParams.__dataclass_fields__:
    _cp = dataclasses.replace(_cp, needs_layout_passes=False)


def _deg_partials(dst_flat):
    @functools.partial(
        pl.kernel,
        out_type=jax.ShapeDtypeStruct((NC, NS, N_PAD), jnp.float32),
        mesh=_mesh,
        scratch_types=[
            pltpu.VMEM((E_PER_TILE,), jnp.int32),
            pltpu.VMEM((N_PAD,), jnp.float32),
        ],
        compiler_params=_cp,
    )
    def k(dst_hbm, degp_hbm, didx, hist):
        cid = lax.axis_index("c")
        sid = lax.axis_index("s")
        ebase = (cid * NS + sid) * E_PER_TILE
        pltpu.sync_copy(dst_hbm.at[pl.ds(ebase, E_PER_TILE)], didx)

        @pl.loop(0, N_PAD, step=16)
        def _(i):
            hist[pl.ds(i, 16)] = jnp.zeros((16,), jnp.float32)

        ones16 = jnp.ones((16,), jnp.float32)

        @pl.loop(0, E_PER_TILE, step=16)
        def _(k):
            plsc.addupdate_scatter(hist, [didx[pl.ds(k, 16)]], ones16)

        pltpu.sync_copy(hist, degp_hbm.at[cid, sid])

    return k(dst_flat)


def _scaled_linear(x, W, degp):
    def body(x_ref, w_ref, d_ref, o_ref, dv_ref):
        h = lax.dot_general(
            x_ref[...],
            w_ref[...],
            (((1,), (1,)), ((), ())),
            preferred_element_type=jnp.float32,
            precision=lax.Precision.HIGHEST,
        )
        deg = jnp.sum(d_ref[0] + d_ref[1], axis=0)
        dinv = jnp.where(deg > 0, lax.rsqrt(jnp.maximum(deg, 1e-12)), 0.0)
        o_ref[...] = h * dinv[:, None]
        dv_ref[...] = dinv[:, None]

    return pl.pallas_call(
        body,
        grid=(N_PAD // BR,),
        in_specs=[
            pl.BlockSpec((BR, D), lambda i: (i, 0)),
            pl.BlockSpec((D, D), lambda i: (0, 0)),
            pl.BlockSpec((NC, NS, BR), lambda i: (0, 0, i)),
        ],
        out_specs=[
            pl.BlockSpec((BR, D), lambda i: (i, 0)),
            pl.BlockSpec((BR, 1), lambda i: (i, 0)),
        ],
        out_shape=[
            jax.ShapeDtypeStruct((N_PAD, D), jnp.float32),
            jax.ShapeDtypeStruct((N_PAD, 1), jnp.float32),
        ],
    )(x, W, degp)


def _gather_scatter(hs, src_flat, dst3):
    @functools.partial(
        pl.kernel,
        out_type=jax.ShapeDtypeStruct((NC, N_PAD, D), jnp.float32),
        mesh=_mesh,
        scratch_types=[
            pltpu.VMEM((N_CHUNKS, CHUNK), jnp.int32),
            pltpu.VMEM((CHUNK,), jnp.int32),
            pltpu.VMEM((CHUNK,), jnp.int32),
            pltpu.VMEM((CHUNK, D), jnp.float32),
            pltpu.VMEM((CHUNK, D), jnp.float32),
            pltpu.VMEM_SHARED((N_PAD, D), jnp.float32),
            pltpu.SemaphoreType.DMA,
            pltpu.SemaphoreType.DMA,
            pltpu.SemaphoreType.DMA,
            pltpu.SemaphoreType.DMA,
        ],
    )
    def k(hs_hbm, src_hbm, dst_hbm, acc_hbm,
          didx, sidxa, sidxb, rowsa, rowsb, acc_sh, gsema, gsemb, isema, isemb):
        cid = lax.axis_index("c")
        sid = lax.axis_index("s")
        wid = cid * NS + sid
        row0 = sid * ROWS_PER_TILE
        ebase = wid * E_PER_TILE

        def sload(c, buf, sem):
            c = jnp.minimum(c, N_CHUNKS - 1)
            pltpu.async_copy(src_hbm.at[pl.ds(ebase + c * CHUNK, CHUNK)],
                             buf, sem)

        def swait(buf, sem):
            pltpu.make_async_copy(src_hbm.at[pl.ds(ebase, CHUNK)],
                                  buf, sem).wait()

        def gstart(buf, idx, sem):
            pltpu.async_copy(hs_hbm.at[idx], buf, sem)

        def gwait(buf, idx, sem):
            pltpu.make_async_copy(hs_hbm.at[idx], buf, sem).wait()

        def scatter(buf, c):
            pltpu.sync_copy(buf, acc_sh.at[didx.at[c]], add=True)

        @pl.loop(0, CHUNK)
        def _(i):
            @pl.loop(0, D, step=16)
            def _(j):
                rowsa[i, pl.ds(j, 16)] = jnp.zeros((16,), jnp.float32)

        @pl.loop(0, ROWS_PER_TILE, step=CHUNK)
        def _(r):
            @pl.loop(0, CHUNK, step=16)
            def _(i):
                sidxa[pl.ds(i, 16)] = lax.iota(jnp.int32, 16) + (row0 + r + i)
            pltpu.sync_copy(rowsa, acc_sh.at[sidxa])
        plsc.subcore_barrier()

        pltpu.sync_copy(dst_hbm.at[wid], didx)

        pltpu.sync_copy(src_hbm.at[pl.ds(ebase, CHUNK)], sidxa)
        gstart(rowsa, sidxa, gsema)
        sload(1, sidxb, isemb)

        @pl.loop(0, N_CHUNKS - 1, step=2)
        def _(j):
            gwait(rowsa, sidxa, gsema)
            sload(j + 2, sidxa, isema)
            swait(sidxb, isemb)
            gstart(rowsb, sidxb, gsemb)
            scatter(rowsa, j)
            gwait(rowsb, sidxb, gsemb)
            sload(j + 3, sidxb, isemb)
            swait(sidxa, isema)
            gstart(rowsa, sidxa, gsema)
            scatter(rowsb, j + 1)

        gwait(rowsa, sidxa, gsema)
        scatter(rowsa, N_CHUNKS - 1)
        swait(sidxb, isemb)

        plsc.subcore_barrier()
        pltpu.sync_copy(
            acc_sh.at[pl.ds(row0, ROWS_PER_TILE)],
            acc_hbm.at[cid, pl.ds(row0, ROWS_PER_TILE)],
        )

    return k(hs, src_flat, dst3)


def _epilogue(accp, dinv):
    EBR = 1000

    def body(a_ref, dv_ref, o_ref):
        o = (a_ref[0] + a_ref[1]) * dv_ref[...]
        o_ref[...] = jnp.maximum(o, 0.0) + o

    return pl.pallas_call(
        body,
        grid=(N_NODES // EBR,),
        in_specs=[
            pl.BlockSpec((NC, EBR, D), lambda i: (0, i, 0)),
            pl.BlockSpec((EBR, 1), lambda i: (i, 0)),
        ],
        out_specs=pl.BlockSpec((EBR, D), lambda i: (i, 0)),
        out_shape=jax.ShapeDtypeStruct((N_NODES, D), jnp.float32),
    )(accp, dinv)


@jax.jit
def kernel(x, edge_index, edge_attr, W):
    src_flat = edge_index[0]
    dst_flat = edge_index[1]
    dst3 = dst_flat.reshape(NW, N_CHUNKS, CHUNK)
    degp = _deg_partials(dst_flat)
    hs, dinv = _scaled_linear(x, W, degp)
    accp = _gather_scatter(hs, src_flat, dst3)
    return _epilogue(accp, dinv)


# speedup vs baseline: 2.0892x; 1.1872x over previous
import dataclasses
import functools

import jax
import jax.numpy as jnp
from jax import lax
from jax.experimental import pallas as pl
from jax.experimental.pallas import tpu as pltpu
from jax.experimental.pallas import tpu_sc as plsc

N_NODES = 10000
N_EDGES = 320000
D = 128

NC = 2
NS = 16
NW = NC * NS
E_PER_TILE = N_EDGES // NW
CHUNK = 80
N_CHUNKS = E_PER_TILE // CHUNK
N_PAD = 10240
ROWS_PER_TILE = N_PAD // NS
BR = 1024

_mesh = plsc.VectorSubcoreMesh(
    core_axis_name="c", subcore_axis_name="s", num_cores=NC, num_subcores=NS
)

_cp = pltpu.CompilerParams()
if "needs_layout_passes" in pltpu.CompilerParams.__dataclass_fields__:
    _cp = dataclasses.replace(_cp, needs_layout_passes=False)


def _deg_partials(dst_flat):
    @functools.partial(
        pl.kernel,
        out_type=jax.ShapeDtypeStruct((NC, NS, N_PAD), jnp.float32),
        mesh=_mesh,
        scratch_types=[
            pltpu.VMEM((E_PER_TILE,), jnp.int32),
            pltpu.VMEM((N_PAD,), jnp.float32),
        ],
        compiler_params=_cp,
    )
    def k(dst_hbm, degp_hbm, didx, hist):
        cid = lax.axis_index("c")
        sid = lax.axis_index("s")
        ebase = (cid * NS + sid) * E_PER_TILE
        pltpu.sync_copy(dst_hbm.at[pl.ds(ebase, E_PER_TILE)], didx)

        @pl.loop(0, N_PAD, step=16)
        def _(i):
            hist[pl.ds(i, 16)] = jnp.zeros((16,), jnp.float32)

        ones16 = jnp.ones((16,), jnp.float32)

        @pl.loop(0, E_PER_TILE, step=16)
        def _(k):
            plsc.addupdate_scatter(hist, [didx[pl.ds(k, 16)]], ones16)

        pltpu.sync_copy(hist, degp_hbm.at[cid, sid])

    return k(dst_flat)


def _scaled_linear(x, W, degp):
    def body(x_ref, w_ref, d_ref, o_ref, dv_ref):
        h = lax.dot_general(
            x_ref[...],
            w_ref[...],
            (((1,), (1,)), ((), ())),
            preferred_element_type=jnp.float32,
            precision=lax.Precision.HIGHEST,
        )
        deg = jnp.sum(d_ref[0] + d_ref[1], axis=0)
        dinv = jnp.where(deg > 0, lax.rsqrt(jnp.maximum(deg, 1e-12)), 0.0)
        o_ref[...] = h * dinv[:, None]
        dv_ref[...] = dinv[:, None]

    return pl.pallas_call(
        body,
        grid=(N_PAD // BR,),
        in_specs=[
            pl.BlockSpec((BR, D), lambda i: (i, 0)),
            pl.BlockSpec((D, D), lambda i: (0, 0)),
            pl.BlockSpec((NC, NS, BR), lambda i: (0, 0, i)),
        ],
        out_specs=[
            pl.BlockSpec((BR, D), lambda i: (i, 0)),
            pl.BlockSpec((BR, 1), lambda i: (i, 0)),
        ],
        out_shape=[
            jax.ShapeDtypeStruct((N_PAD, D), jnp.float32),
            jax.ShapeDtypeStruct((N_PAD, 1), jnp.float32),
        ],
    )(x, W, degp)


def _gather_scatter(hs, src_flat, dst3):
    @functools.partial(
        pl.kernel,
        out_type=jax.ShapeDtypeStruct((NC, N_PAD, D), jnp.float32),
        mesh=_mesh,
        scratch_types=[
            pltpu.VMEM((N_CHUNKS, CHUNK), jnp.int32),
            pltpu.VMEM((CHUNK,), jnp.int32),
            pltpu.VMEM((CHUNK,), jnp.int32),
            pltpu.VMEM((CHUNK, D), jnp.float32),
            pltpu.VMEM((CHUNK, D), jnp.float32),
            pltpu.VMEM_SHARED((N_PAD, D), jnp.float32),
            pltpu.SemaphoreType.DMA,
            pltpu.SemaphoreType.DMA,
            pltpu.SemaphoreType.DMA,
            pltpu.SemaphoreType.DMA,
        ],
    )
    def k(hs_hbm, src_hbm, dst_hbm, acc_hbm,
          didx, sidxa, sidxb, rowsa, rowsb, acc_sh, gsema, gsemb, isema, isemb):
        cid = lax.axis_index("c")
        sid = lax.axis_index("s")
        wid = cid * NS + sid
        row0 = sid * ROWS_PER_TILE
        ebase = wid * E_PER_TILE

        def sload(c, buf, sem):
            c = jnp.minimum(c, N_CHUNKS - 1)
            pltpu.async_copy(src_hbm.at[pl.ds(ebase + c * CHUNK, CHUNK)],
                             buf, sem)

        def swait(buf, sem):
            pltpu.make_async_copy(src_hbm.at[pl.ds(ebase, CHUNK)],
                                  buf, sem).wait()

        def gstart(buf, idx, sem):
            pltpu.async_copy(hs_hbm.at[idx], buf, sem)

        def gwait(buf, idx, sem):
            pltpu.make_async_copy(hs_hbm.at[idx], buf, sem).wait()

        def scatter(buf, c):
            pltpu.sync_copy(buf, acc_sh.at[didx.at[c]], add=True)

        @pl.loop(0, CHUNK)
        def _(i):
            @pl.loop(0, D, step=16)
            def _(j):
                rowsa[i, pl.ds(j, 16)] = jnp.zeros((16,), jnp.float32)

        @pl.loop(0, ROWS_PER_TILE, step=CHUNK)
        def _(r):
            @pl.loop(0, CHUNK, step=16)
            def _(i):
                sidxa[pl.ds(i, 16)] = lax.iota(jnp.int32, 16) + (row0 + r + i)
            pltpu.sync_copy(rowsa, acc_sh.at[sidxa])
        plsc.subcore_barrier()

        pltpu.sync_copy(dst_hbm.at[wid], didx)

        pltpu.sync_copy(src_hbm.at[pl.ds(ebase, CHUNK)], sidxa)
        gstart(rowsa, sidxa, gsema)
        pltpu.sync_copy(src_hbm.at[pl.ds(ebase + CHUNK, CHUNK)], sidxb)
        gstart(rowsb, sidxb, gsemb)

        @pl.loop(0, N_CHUNKS - 1, step=2)
        def _(j):
            gwait(rowsa, sidxa, gsema)
            sload(j + 2, sidxa, isema)
            scatter(rowsa, j)
            swait(sidxa, isema)
            gstart(rowsa, sidxa, gsema)
            gwait(rowsb, sidxb, gsemb)
            sload(j + 3, sidxb, isemb)
            scatter(rowsb, j + 1)
            swait(sidxb, isemb)
            gstart(rowsb, sidxb, gsemb)

        gwait(rowsa, sidxa, gsema)
        scatter(rowsa, N_CHUNKS - 1)
        gwait(rowsb, sidxb, gsemb)

        plsc.subcore_barrier()
        pltpu.sync_copy(
            acc_sh.at[pl.ds(row0, ROWS_PER_TILE)],
            acc_hbm.at[cid, pl.ds(row0, ROWS_PER_TILE)],
        )

    return k(hs, src_flat, dst3)


def _epilogue(accp, dinv):
    EBR = 1000

    def body(a_ref, dv_ref, o_ref):
        o = (a_ref[0] + a_ref[1]) * dv_ref[...]
        o_ref[...] = jnp.maximum(o, 0.0) + o

    return pl.pallas_call(
        body,
        grid=(N_NODES // EBR,),
        in_specs=[
            pl.BlockSpec((NC, EBR, D), lambda i: (0, i, 0)),
            pl.BlockSpec((EBR, 1), lambda i: (i, 0)),
        ],
        out_specs=pl.BlockSpec((EBR, D), lambda i: (i, 0)),
        out_shape=jax.ShapeDtypeStruct((N_NODES, D), jnp.float32),
    )(accp, dinv)


@jax.jit
def kernel(x, edge_index, edge_attr, W):
    src_flat = edge_index[0]
    dst_flat = edge_index[1]
    dst3 = dst_flat.reshape(NW, N_CHUNKS, CHUNK)
    degp = _deg_partials(dst_flat)
    hs, dinv = _scaled_linear(x, W, degp)
    accp = _gather_scatter(hs, src_flat, dst3)
    return _epilogue(accp, dinv)
